# baseline (speedup 1.0000x reference)
import functools

import jax
import jax.numpy as jnp
from jax import lax
from jax.experimental import pallas as pl
from jax.experimental.pallas import tpu as pltpu
from jax.experimental.pallas import tpu_sc as plsc

K_TOT = 32
DIL = 2
B = 4
N = 4096
D = 64
QB = 512
CH = 64
NCH = N // CH

NWORK = 32
SLICE = N // 2
RPW = SLICE // NWORK
G = 8
NEG_INF = float("-inf")


def _dist_body(xq_ref, xk_ref, out_ref, mx_ref):
    q = xq_ref[...]
    k = xk_ref[...]
    inner = jax.lax.dot_general(
        q, k, (((1,), (1,)), ((), ())),
        preferred_element_type=jnp.float32,
    )
    x_inner = -2.0 * inner
    qsq = jnp.sum(q * q, axis=-1, keepdims=True)
    ksq = jnp.sum(k * k, axis=-1, keepdims=True)
    neg = -(qsq + x_inner + ksq.T)
    out_ref[...] = jnp.reshape(neg, (QB * (N // 128), 128))
    mx = jnp.max(jnp.reshape(neg, (QB, NCH, CH)), axis=2)
    mx_ref[...] = jnp.reshape(mx, (QB * NCH // 128, 128))


def _neg_adj_slice(xq, xk):
    return pl.pallas_call(
        _dist_body,
        grid=(SLICE // QB,),
        in_specs=[
            pl.BlockSpec((QB, D), lambda i: (i, 0)),
            pl.BlockSpec((N, D), lambda i: (0, 0)),
        ],
        out_specs=[
            pl.BlockSpec((QB * (N // 128), 128), lambda i: (i, 0)),
            pl.BlockSpec((QB * NCH // 128, 128), lambda i: (i, 0)),
        ],
        out_shape=[
            jax.ShapeDtypeStruct((SLICE * (N // 128), 128), jnp.float32),
            jax.ShapeDtypeStruct((SLICE * NCH // 128, 128), jnp.float32),
        ],
    )(xq, xk)


_MESH = plsc.VectorSubcoreMesh(core_axis_name="c", subcore_axis_name="s")
_VPR = N // 16


def _merge32(av, ai, bv, bi):
    rbv1, rbi1 = lax.rev(bv[1], (0,)), lax.rev(bi[1], (0,))
    rbv0, rbi0 = lax.rev(bv[0], (0,)), lax.rev(bi[0], (0,))
    k0 = av[0] >= rbv1
    m0v = jnp.where(k0, av[0], rbv1)
    m0i = jnp.where(k0, ai[0], rbi1)
    k1 = av[1] >= rbv0
    m1v = jnp.where(k1, av[1], rbv0)
    m1i = jnp.where(k1, ai[1], rbi0)
    c = m0v >= m1v
    hv = jnp.where(c, m0v, m1v)
    hx = jnp.where(c, m0i, m1i)
    lv = jnp.where(c, m1v, m0v)
    lx = jnp.where(c, m1i, m0i)
    s0v, s0i = plsc.sort_key_val(hv, hx, descending=True)
    s1v, s1i = plsc.sort_key_val(lv, lx, descending=True)
    return (s0v, s1v), (s0i, s1i)


@functools.partial(
    pl.kernel,
    out_type=(
        jax.ShapeDtypeStruct((SLICE * K_TOT,), jnp.float32),
        jax.ShapeDtypeStruct((SLICE * K_TOT,), jnp.int32),
    ),
    mesh=_MESH,
    compiler_params=pltpu.CompilerParams(needs_layout_passes=False),
    scratch_types=[
        pltpu.VMEM((2 * G * N,), jnp.float32),
        pltpu.VMEM((RPW * NCH,), jnp.float32),
        pltpu.VMEM((_VPR,), jnp.int32),
        pltpu.VMEM((_VPR + 16,), jnp.int32),
        pltpu.VMEM((N + 16,), jnp.int32),
        pltpu.VMEM((RPW * K_TOT,), jnp.float32),
        pltpu.VMEM((RPW * K_TOT,), jnp.int32),
        pltpu.SemaphoreType.DMA,
        pltpu.SemaphoreType.DMA,
    ],
)
def _topk_sc(neg_hbm, mx_hbm, val_out, idx_out,
             inbuf, mxall, flags, hits, cand_i, outv, outi, sem0, sem1):
    cid = lax.axis_index("c")
    sid = lax.axis_index("s")
    wid = sid * 2 + cid
    row0 = wid * RPW
    lane = lax.iota(jnp.int32, 16)
    ninf16 = jnp.full((16,), NEG_INF, jnp.float32)
    zero16 = jnp.zeros((16,), jnp.int32)

    def do_row(rbase, slot):
        mbase = slot * NCH
        s0, _ = plsc.sort_key_val(
            mxall[pl.ds(mbase, 16)], zero16, descending=True)
        s1, _ = plsc.sort_key_val(
            mxall[pl.ds(mbase + 16, 16)], zero16, descending=True)
        s2, _ = plsc.sort_key_val(
            mxall[pl.ds(mbase + 32, 16)], zero16, descending=True)
        s3, _ = plsc.sort_key_val(
            mxall[pl.ds(mbase + 48, 16)], zero16, descending=True)
        a, _ = _merge32((s0, ninf16), (zero16, zero16),
                        (s1, ninf16), (zero16, zero16))
        b, _ = _merge32((s2, ninf16), (zero16, zero16),
                        (s3, ninf16), (zero16, zero16))
        w, _ = _merge32(a, (zero16, zero16), b, (zero16, zero16))
        t = jnp.min(w[1])

        def scan_b1(w1, _):
            fl = jnp.zeros((16,), jnp.int32)
            for u in range(16):
                j = w1 * 16 + u
                v = inbuf[pl.ds(pl.multiple_of(rbase + j * 16, 16), 16)]
                mask = v >= t
                c = plsc.all_reduce_population_count(mask)
                fl = jnp.where(lane == u, c, fl)
            flags[pl.ds(pl.multiple_of(w1 * 16, 16), 16)] = fl
            return _

        lax.fori_loop(0, _VPR // 16, scan_b1, 0)

        def scan_b2(w2, off):
            fl = flags[pl.ds(pl.multiple_of(w2 * 16, 16), 16)]
            mask = fl > 0
            plsc.store_compressed(hits.at[pl.ds(off, 16)], lane + w2 * 16,
                                  mask=mask)
            return off + plsc.all_reduce_population_count(mask)[0]

        nhit = lax.fori_loop(0, _VPR // 16, scan_b2, jnp.int32(0))

        def scan_b3(k, off):
            hv = hits[pl.ds(pl.multiple_of(k * 16, 16), 16)]
            for u in range(16):
                valid = (k * 16 + u) < nhit
                j = jnp.bitwise_and(hv[u], _VPR - 1)
                v = inbuf[pl.ds(pl.multiple_of(rbase + j * 16, 16), 16)]
                mask = jnp.logical_and(v >= t, valid)
                idxv = lane + j * 16
                plsc.store_compressed(cand_i.at[pl.ds(off, 16)], idxv,
                                      mask=mask)
                off = off + plsc.all_reduce_population_count(mask)[0]
            return off

        nbat = (nhit + 15) // 16
        cnt = lax.fori_loop(0, nbat, scan_b3, jnp.int32(0))
        cand_i[pl.ds(cnt, 16)] = zero16

        def merge(j, carry):
            b0v, b0i, b1v, b1i = carry
            ci = cand_i[pl.ds(pl.multiple_of(j * 16, 16), 16)]
            cv = plsc.load_gather(inbuf, [ci + rbase])
            valid = (lane + j * 16) < cnt
            cv = jnp.where(valid, cv, NEG_INF)
            cv, ci = plsc.sort_key_val(cv, ci, descending=True)
            (b0v, b1v), (b0i, b1i) = _merge32(
                (b0v, b1v), (b0i, b1i),
                (cv, ninf16), (ci, zero16))
            return b0v, b0i, b1v, b1i

        nch = (cnt + 15) // 16
        b0v, b0i, b1v, b1i = lax.fori_loop(
            0, nch, merge, (ninf16, zero16, ninf16, zero16))

        obase = slot * K_TOT
        outv[pl.ds(obase, 16)] = b0v
        outv[pl.ds(obase + 16, 16)] = b1v
        outi[pl.ds(obase, 16)] = b0i
        outi[pl.ds(obase + 16, 16)] = b1i

    pltpu.sync_copy(mx_hbm.at[pl.ds(row0 * NCH, RPW * NCH)], mxall)

    half = G * N

    def _src(g):
        return neg_hbm.at[pl.ds((row0 + g * G) * N, G * N)]

    def _buf(p):
        return inbuf.at[pl.ds(p * half, half)]

    def _process(g, p):
        def row_body(rr, __):
            do_row(p * half + rr * N, g * G + rr)
            return __

        lax.fori_loop(0, G, row_body, 0)

    NGRP = RPW // G
    pltpu.async_copy(_src(0), _buf(0), sem0)

    def pair(i, _):
        g0 = i * 2
        pltpu.async_copy(_src(g0 + 1), _buf(1), sem1)
        pltpu.make_async_copy(_src(g0), _buf(0), sem0).wait()
        _process(g0, 0)

        @pl.when(g0 + 2 < NGRP)
        def _start_next():
            pltpu.async_copy(_src(g0 + 2), _buf(0), sem0)

        pltpu.make_async_copy(_src(g0 + 1), _buf(1), sem1).wait()
        _process(g0 + 1, 1)
        return _

    lax.fori_loop(0, NGRP // 2, pair, 0)
    pltpu.sync_copy(outv, val_out.at[pl.ds(row0 * K_TOT, RPW * K_TOT)])
    pltpu.sync_copy(outi, idx_out.at[pl.ds(row0 * K_TOT, RPW * K_TOT)])


def kernel(x, batch):
    del batch
    xb = x.reshape(B, N, D)
    vals, idxs = [], []
    for b in range(B):
        for h in range(N // SLICE):
            xq = lax.slice_in_dim(xb[b], h * SLICE, (h + 1) * SLICE, axis=0)
            neg_flat, mx_flat = _neg_adj_slice(xq, xb[b])
            v, i = _topk_sc(neg_flat.reshape(SLICE * N),
                            mx_flat.reshape(SLICE * NCH))
            vals.append(v)
            idxs.append(i)
    val = jnp.stack(vals).reshape(1, -1)
    start = (jnp.arange(B, dtype=jnp.int32) * N).reshape(B, 1, 1)
    nn_idx = (jnp.stack(idxs).reshape(B, N, K_TOT) + start).reshape(1, -1)
    center = jnp.repeat(jnp.arange(B * N, dtype=jnp.int32), K_TOT).reshape(1, -1)
    edge_index = jnp.concatenate([nn_idx, center], axis=0)[:, ::DIL]
    return edge_index, val


# speedup vs baseline: 1.0035x; 1.0035x over previous
import functools

import jax
import jax.numpy as jnp
from jax import lax
from jax.experimental import pallas as pl
from jax.experimental.pallas import tpu as pltpu
from jax.experimental.pallas import tpu_sc as plsc

K_TOT = 32
DIL = 2
B = 4
N = 4096
D = 64
QB = 512
CH = 64
NCH = N // CH

NWORK = 32
RPW = N // NWORK
G = 8
NEG_INF = float("-inf")


def _dist_body(xq_ref, xk_ref, out_ref, mx_ref):
    q = xq_ref[...]
    k = xk_ref[...]
    inner = jax.lax.dot_general(
        q, k, (((1,), (1,)), ((), ())),
        preferred_element_type=jnp.float32,
    )
    x_inner = -2.0 * inner
    qsq = jnp.sum(q * q, axis=-1, keepdims=True)
    ksq = jnp.sum(k * k, axis=-1, keepdims=True)
    neg = -(qsq + x_inner + ksq.T)
    out_ref[...] = jnp.reshape(neg, (QB * (N // 128), 128))
    mx = jnp.max(jnp.reshape(neg, (QB, NCH, CH)), axis=2)
    mx_ref[...] = jnp.reshape(mx, (QB * NCH // 128, 128))


def _neg_adj_slice(xs):
    return pl.pallas_call(
        _dist_body,
        grid=(N // QB,),
        in_specs=[
            pl.BlockSpec((QB, D), lambda i: (i, 0)),
            pl.BlockSpec((N, D), lambda i: (0, 0)),
        ],
        out_specs=[
            pl.BlockSpec((QB * (N // 128), 128), lambda i: (i, 0)),
            pl.BlockSpec((QB * NCH // 128, 128), lambda i: (i, 0)),
        ],
        out_shape=[
            jax.ShapeDtypeStruct((N * (N // 128), 128), jnp.float32),
            jax.ShapeDtypeStruct((N * NCH // 128, 128), jnp.float32),
        ],
    )(xs, xs)


_MESH = plsc.VectorSubcoreMesh(core_axis_name="c", subcore_axis_name="s")
_VPR = N // 16


def _merge32(av, ai, bv, bi):
    rbv1, rbi1 = lax.rev(bv[1], (0,)), lax.rev(bi[1], (0,))
    rbv0, rbi0 = lax.rev(bv[0], (0,)), lax.rev(bi[0], (0,))
    k0 = av[0] >= rbv1
    m0v = jnp.where(k0, av[0], rbv1)
    m0i = jnp.where(k0, ai[0], rbi1)
    k1 = av[1] >= rbv0
    m1v = jnp.where(k1, av[1], rbv0)
    m1i = jnp.where(k1, ai[1], rbi0)
    c = m0v >= m1v
    hv = jnp.where(c, m0v, m1v)
    hx = jnp.where(c, m0i, m1i)
    lv = jnp.where(c, m1v, m0v)
    lx = jnp.where(c, m1i, m0i)
    s0v, s0i = plsc.sort_key_val(hv, hx, descending=True)
    s1v, s1i = plsc.sort_key_val(lv, lx, descending=True)
    return (s0v, s1v), (s0i, s1i)


@functools.partial(
    pl.kernel,
    out_type=(
        jax.ShapeDtypeStruct((N * K_TOT,), jnp.float32),
        jax.ShapeDtypeStruct((N * K_TOT,), jnp.int32),
    ),
    mesh=_MESH,
    compiler_params=pltpu.CompilerParams(needs_layout_passes=False),
    scratch_types=[
        pltpu.VMEM((2 * G * N,), jnp.float32),
        pltpu.VMEM((RPW * NCH,), jnp.float32),
        pltpu.VMEM((_VPR,), jnp.int32),
        pltpu.VMEM((_VPR + 16,), jnp.int32),
        pltpu.VMEM((N + 16,), jnp.int32),
        pltpu.VMEM((RPW * K_TOT,), jnp.float32),
        pltpu.VMEM((RPW * K_TOT,), jnp.int32),
        pltpu.SemaphoreType.DMA,
        pltpu.SemaphoreType.DMA,
    ],
)
def _topk_sc(neg_hbm, mx_hbm, val_out, idx_out,
             inbuf, mxall, flags, hits, cand_i, outv, outi, sem0, sem1):
    cid = lax.axis_index("c")
    sid = lax.axis_index("s")
    wid = sid * 2 + cid
    row0 = wid * RPW
    lane = lax.iota(jnp.int32, 16)
    ninf16 = jnp.full((16,), NEG_INF, jnp.float32)
    zero16 = jnp.zeros((16,), jnp.int32)

    def do_row(rbase, slot):
        mbase = slot * NCH
        s0, _ = plsc.sort_key_val(
            mxall[pl.ds(mbase, 16)], zero16, descending=True)
        s1, _ = plsc.sort_key_val(
            mxall[pl.ds(mbase + 16, 16)], zero16, descending=True)
        s2, _ = plsc.sort_key_val(
            mxall[pl.ds(mbase + 32, 16)], zero16, descending=True)
        s3, _ = plsc.sort_key_val(
            mxall[pl.ds(mbase + 48, 16)], zero16, descending=True)
        a, _ = _merge32((s0, ninf16), (zero16, zero16),
                        (s1, ninf16), (zero16, zero16))
        b, _ = _merge32((s2, ninf16), (zero16, zero16),
                        (s3, ninf16), (zero16, zero16))
        w, _ = _merge32(a, (zero16, zero16), b, (zero16, zero16))
        t = jnp.min(w[1])

        def scan_b1(w1, _):
            fl = jnp.zeros((16,), jnp.int32)
            for u in range(16):
                j = w1 * 16 + u
                v = inbuf[pl.ds(pl.multiple_of(rbase + j * 16, 16), 16)]
                mask = v >= t
                c = plsc.all_reduce_population_count(mask)
                fl = jnp.where(lane == u, c, fl)
            flags[pl.ds(pl.multiple_of(w1 * 16, 16), 16)] = fl
            return _

        lax.fori_loop(0, _VPR // 16, scan_b1, 0)

        def scan_b2(w2, off):
            fl = flags[pl.ds(pl.multiple_of(w2 * 16, 16), 16)]
            mask = fl > 0
            plsc.store_compressed(hits.at[pl.ds(off, 16)], lane + w2 * 16,
                                  mask=mask)
            return off + plsc.all_reduce_population_count(mask)[0]

        nhit = lax.fori_loop(0, _VPR // 16, scan_b2, jnp.int32(0))

        def scan_b3(k, off):
            hv = hits[pl.ds(pl.multiple_of(k * 16, 16), 16)]
            for u in range(16):
                valid = (k * 16 + u) < nhit
                j = jnp.bitwise_and(hv[u], _VPR - 1)
                v = inbuf[pl.ds(pl.multiple_of(rbase + j * 16, 16), 16)]
                mask = jnp.logical_and(v >= t, valid)
                idxv = lane + j * 16
                plsc.store_compressed(cand_i.at[pl.ds(off, 16)], idxv,
                                      mask=mask)
                off = off + plsc.all_reduce_population_count(mask)[0]
            return off

        nbat = (nhit + 15) // 16
        cnt = lax.fori_loop(0, nbat, scan_b3, jnp.int32(0))
        cand_i[pl.ds(cnt, 16)] = zero16

        def merge(j, carry):
            b0v, b0i, b1v, b1i = carry
            ci = cand_i[pl.ds(pl.multiple_of(j * 16, 16), 16)]
            cv = plsc.load_gather(inbuf, [ci + rbase])
            valid = (lane + j * 16) < cnt
            cv = jnp.where(valid, cv, NEG_INF)
            cv, ci = plsc.sort_key_val(cv, ci, descending=True)
            (b0v, b1v), (b0i, b1i) = _merge32(
                (b0v, b1v), (b0i, b1i),
                (cv, ninf16), (ci, zero16))
            return b0v, b0i, b1v, b1i

        nch = (cnt + 15) // 16
        b0v, b0i, b1v, b1i = lax.fori_loop(
            0, nch, merge, (ninf16, zero16, ninf16, zero16))

        obase = slot * K_TOT
        outv[pl.ds(obase, 16)] = b0v
        outv[pl.ds(obase + 16, 16)] = b1v
        outi[pl.ds(obase, 16)] = b0i
        outi[pl.ds(obase + 16, 16)] = b1i

    pltpu.sync_copy(mx_hbm.at[pl.ds(row0 * NCH, RPW * NCH)], mxall)

    half = G * N

    def _src(g):
        return neg_hbm.at[pl.ds((row0 + g * G) * N, G * N)]

    def _buf(p):
        return inbuf.at[pl.ds(p * half, half)]

    def _process(g, p):
        def row_body(rr, __):
            do_row(p * half + rr * N, g * G + rr)
            return __

        lax.fori_loop(0, G, row_body, 0)

    NGRP = RPW // G
    pltpu.async_copy(_src(0), _buf(0), sem0)

    def pair(i, _):
        g0 = i * 2
        pltpu.async_copy(_src(g0 + 1), _buf(1), sem1)
        pltpu.make_async_copy(_src(g0), _buf(0), sem0).wait()
        _process(g0, 0)

        @pl.when(g0 + 2 < NGRP)
        def _start_next():
            pltpu.async_copy(_src(g0 + 2), _buf(0), sem0)

        pltpu.make_async_copy(_src(g0 + 1), _buf(1), sem1).wait()
        _process(g0 + 1, 1)
        return _

    lax.fori_loop(0, NGRP // 2, pair, 0)
    pltpu.sync_copy(outv, val_out.at[pl.ds(row0 * K_TOT, RPW * K_TOT)])
    pltpu.sync_copy(outi, idx_out.at[pl.ds(row0 * K_TOT, RPW * K_TOT)])


def kernel(x, batch):
    del batch
    xb = x.reshape(B, N, D)
    vals, idxs = [], []
    for b in range(B):
        neg_flat, mx_flat = _neg_adj_slice(xb[b])
        v, i = _topk_sc(neg_flat.reshape(N * N), mx_flat.reshape(N * NCH))
        vals.append(v)
        idxs.append(i)
    val = jnp.stack(vals).reshape(1, -1)
    start = (jnp.arange(B, dtype=jnp.int32) * N).reshape(B, 1, 1)
    nn_idx = (jnp.stack(idxs).reshape(B, N, K_TOT) + start).reshape(1, -1)
    center = jnp.repeat(jnp.arange(B * N, dtype=jnp.int32), K_TOT).reshape(1, -1)
    edge_index = jnp.concatenate([nn_idx, center], axis=0)[:, ::DIL]
    return edge_index, val
